# baseline (speedup 1.0000x reference)
import jax
import jax.numpy as jnp
from jax import lax
from jax.experimental import pallas as pl
from jax.experimental.pallas import tpu as pltpu
from jax.experimental.pallas import tpu_sc as plsc

_BATCH = 16384
_DIM = 64
_GAMMA = 12.0
_NW = 32
_BPW = _BATCH // _NW
_CHUNK = 64
_NCHUNK = _BPW // _CHUNK
_GROUPS = _CHUNK // 16


def _tec_body(heads_h, rels_h, tails_h, nheads_h, ntails_h, ent_h, rel_h,
              out_h,
              h_idx, r_idx, t_idx, nh_idx, nt_idx, row_idx,
              h_rows, r_rows, t_rows, nh_rows, nt_rows,
              out_stage, sems):
    wid = lax.axis_index("s") * 2 + lax.axis_index("c")
    base = wid * _BPW

    pltpu.sync_copy(heads_h.at[pl.ds(base, _BPW)], h_idx)
    pltpu.sync_copy(rels_h.at[pl.ds(base, _BPW)], r_idx)
    pltpu.sync_copy(tails_h.at[pl.ds(base, _BPW)], t_idx)
    pltpu.sync_copy(nheads_h.at[pl.ds(base, _BPW)], nh_idx)
    pltpu.sync_copy(ntails_h.at[pl.ds(base, _BPW)], nt_idx)

    idx_bufs = (h_idx, r_idx, t_idx, nh_idx, nt_idx)

    def shift_body(i, _):
        for b in range(5):
            v = idx_bufs[b][pl.ds(i * 16, 16)]
            row_idx[b, pl.ds(i * 16, 16)] = jax.lax.shift_right_logical(v, 1)
        return 0

    lax.fori_loop(0, _BPW // 16, shift_body, 0, unroll=2)

    def start_chunk(c, buf_par):
        off = c * _CHUNK
        sem = sems.at[buf_par]
        srcs = (ent_h, rel_h, ent_h, ent_h, ent_h)
        dsts = (h_rows, r_rows, t_rows, nh_rows, nt_rows)
        return [
            pltpu.async_copy(
                srcs[b].at[row_idx.at[b, pl.ds(off, _CHUNK)]],
                dsts[b].at[buf_par], sem)
            for b in range(5)
        ]

    zero = jnp.zeros((16,), jnp.float32)
    v_loss, v_pos, v_neg = zero, zero, zero
    gamma = zero + _GAMMA
    iota = lax.iota(jnp.int32, 16)
    one = jnp.full((16,), 1, jnp.int32)

    descs = [None, None]
    descs[0] = start_chunk(0, 0)

    for c in range(_NCHUNK):
        buf_par = c % 2
        if c + 1 < _NCHUNK:
            descs[(c + 1) % 2] = start_chunk(c + 1, (c + 1) % 2)
        for d in descs[buf_par]:
            d.wait()

        hb, rb, tb = h_rows.at[buf_par], r_rows.at[buf_par], t_rows.at[buf_par]
        nhb, ntb = nh_rows.at[buf_par], nt_rows.at[buf_par]

        for g in range(_GROUPS):
            goff = c * _CHUNK + g * 16
            rows = iota + g * 16
            cols = [(idx_bufs[b][pl.ds(goff, 16)] & one) * 64
                    for b in range(5)]

            def dim_body(d, acc, hb=hb, rb=rb, tb=tb, nhb=nhb, ntb=ntb,
                         rows=rows, cols=cols):
                ap, an = acc
                h = plsc.load_gather(hb, [rows, cols[0] + d])
                r = plsc.load_gather(rb, [rows, cols[1] + d])
                t = plsc.load_gather(tb, [rows, cols[2] + d])
                nh = plsc.load_gather(nhb, [rows, cols[3] + d])
                nt = plsc.load_gather(ntb, [rows, cols[4] + d])
                ap = ap + jnp.abs(h + r - t)
                an = an + jnp.abs(nh + r - nt)
                return ap, an

            ap, an = lax.fori_loop(0, _DIM, dim_body, (zero, zero), unroll=2)
            v_loss = v_loss + jnp.maximum(gamma + ap - an, 0.0)
            v_pos = v_pos + ap
            v_neg = v_neg + an

    out_stage[0, pl.ds(0, 16)] = v_loss
    out_stage[1, pl.ds(0, 16)] = v_pos
    out_stage[2, pl.ds(0, 16)] = v_neg
    pltpu.sync_copy(out_stage, out_h.at[wid])


@jax.jit
def _transe_sc(heads, relations, tails, negative_heads, negative_tails,
               entity_emb, relation_emb):
    ent2 = entity_emb.reshape(entity_emb.shape[0] // 2, 2 * _DIM)
    rel2 = relation_emb.reshape(relation_emb.shape[0] // 2, 2 * _DIM)
    mesh = plsc.VectorSubcoreMesh(core_axis_name="c", subcore_axis_name="s")
    partials = pl.kernel(
        _tec_body,
        out_type=jax.ShapeDtypeStruct((_NW, 8, 128), jnp.float32),
        mesh=mesh,
        compiler_params=pltpu.CompilerParams(needs_layout_passes=False,
                                             use_tc_tiling_on_sc=True),
        scratch_types=[
            pltpu.VMEM((_BPW,), jnp.int32),
            pltpu.VMEM((_BPW,), jnp.int32),
            pltpu.VMEM((_BPW,), jnp.int32),
            pltpu.VMEM((_BPW,), jnp.int32),
            pltpu.VMEM((_BPW,), jnp.int32),
            pltpu.VMEM((5, _BPW), jnp.int32),
            pltpu.VMEM((2, _CHUNK, 2 * _DIM), jnp.float32),
            pltpu.VMEM((2, _CHUNK, 2 * _DIM), jnp.float32),
            pltpu.VMEM((2, _CHUNK, 2 * _DIM), jnp.float32),
            pltpu.VMEM((2, _CHUNK, 2 * _DIM), jnp.float32),
            pltpu.VMEM((2, _CHUNK, 2 * _DIM), jnp.float32),
            pltpu.VMEM((8, 128), jnp.float32),
            pltpu.SemaphoreType.DMA((2,)),
        ],
    )(heads, relations, tails, negative_heads, negative_tails, ent2, rel2)
    sums = jnp.sum(partials[:, 0:3, 0:16], axis=(0, 2))
    inv_b = 1.0 / _BATCH
    return sums[0] * inv_b, sums[1] * inv_b, sums[2] * inv_b


def kernel(heads, relations, tails, negative_heads, negative_tails,
           entity_emb, relation_emb):
    return _transe_sc(heads.astype(jnp.int32), relations.astype(jnp.int32),
                      tails.astype(jnp.int32),
                      negative_heads.astype(jnp.int32),
                      negative_tails.astype(jnp.int32),
                      entity_emb, relation_emb)
